# baseline (speedup 1.0000x reference)
import functools

import jax
import jax.numpy as jnp
from jax.experimental import pallas as pl
from jax.experimental.pallas import tpu as pltpu


def _leaky(x):
    return jnp.where(x >= 0, x, 0.01 * x)


def _gru_body(T, F, H, B, x_ref, wih_ref, whh_ref, bih_ref, bhh_ref, w1_ref,
              o_ref):
    x = x_ref[0]
    wih = wih_ref[...]
    whh = whh_ref[...]
    bih = bih_ref[0]
    bhh = bhh_ref[0]
    h = None
    for t in range(T):
        xt = x[:, t * F:(t + 1) * F]
        gi = jnp.dot(xt, wih, preferred_element_type=jnp.float32) + bih
        if h is None:
            gh = jnp.broadcast_to(bhh, gi.shape)
        else:
            gh = jnp.dot(h.astype(jnp.bfloat16), whh,
                         preferred_element_type=jnp.float32) + bhh
        r = 0.5 * jnp.tanh(0.5 * (gi[:, :H] + gh[:, :H])) + 0.5
        z = 0.5 * jnp.tanh(0.5 * (gi[:, H:2 * H] + gh[:, H:2 * H])) + 0.5
        n = jnp.tanh(gi[:, 2 * H:] + r * gh[:, 2 * H:])
        if h is None:
            h = (1.0 - z) * n
        else:
            h = (1.0 - z) * n + z * h
    y1 = jnp.dot(h.astype(jnp.bfloat16), w1_ref[...],
                 preferred_element_type=jnp.float32)
    o_ref[...] = y1.astype(jnp.bfloat16)


def _gcn_fused_body(B, H, BM, a_ref, y1_ref, b1_ref, w2_ref, b2_ref,
                    wlin_ref, blin_ref, o_ref, y2_ref):
    p = pl.program_id(0)
    j = pl.program_id(1)

    @pl.when(p == 0)
    def _layer1():
        u = jnp.dot(a_ref[...], y1_ref[...], preferred_element_type=jnp.float32)
        t2 = _leaky(u + b1_ref[0])
        w2 = w2_ref[...]
        for b in range(B):
            yb = jnp.dot(t2[:, b * H:(b + 1) * H].astype(jnp.bfloat16), w2,
                         preferred_element_type=jnp.float32)
            y2_ref[pl.ds(j * BM, BM), b * H:(b + 1) * H] = yb.astype(jnp.bfloat16)

    @pl.when(p == 1)
    def _layer2():
        v = jnp.dot(a_ref[...], y2_ref[...], preferred_element_type=jnp.float32)
        t3 = _leaky(v + b2_ref[0])
        wlin = wlin_ref[...]
        blin = blin_ref[0]
        for b in range(B):
            ob = jnp.dot(t3[:, b * H:(b + 1) * H].astype(jnp.bfloat16), wlin,
                         preferred_element_type=jnp.float32) + blin
            o_ref[b] = ob


def kernel(A, X, gru_Wih, gru_Whh, gru_bih, gru_bhh, W1, b1, W2, b2, Wlin, blin):
    B, N, T, F = X.shape
    H = gru_Whh.shape[1]
    T_OUT = Wlin.shape[1]

    Xr = X.reshape(B, N, T * F).astype(jnp.bfloat16)
    Abf = A.astype(jnp.bfloat16)
    wih_t = gru_Wih.T.astype(jnp.bfloat16)
    whh_t = gru_Whh.T.astype(jnp.bfloat16)
    bih2 = gru_bih.reshape(1, -1)
    bhh2 = gru_bhh.reshape(1, -1)
    b1t = jnp.tile(b1, B).reshape(1, B * H)
    b2t = jnp.tile(b2, B).reshape(1, B * H)

    BM_G = min(N, 2048)
    BM_A = min(N, 256)

    y1 = pl.pallas_call(
        functools.partial(_gru_body, T, F, H, B),
        grid=(B, N // BM_G),
        in_specs=[
            pl.BlockSpec((1, BM_G, T * F), lambda b, j: (b, j, 0)),
            pl.BlockSpec((F, 3 * H), lambda b, j: (0, 0)),
            pl.BlockSpec((H, 3 * H), lambda b, j: (0, 0)),
            pl.BlockSpec((1, 3 * H), lambda b, j: (0, 0)),
            pl.BlockSpec((1, 3 * H), lambda b, j: (0, 0)),
            pl.BlockSpec((H, H), lambda b, j: (0, 0)),
        ],
        out_specs=pl.BlockSpec((BM_G, H), lambda b, j: (j, b)),
        out_shape=jax.ShapeDtypeStruct((N, B * H), jnp.bfloat16),
        compiler_params=pltpu.CompilerParams(
            dimension_semantics=("parallel", "parallel")),
    )(Xr, wih_t, whh_t, bih2, bhh2, W1.astype(jnp.bfloat16))

    out = pl.pallas_call(
        functools.partial(_gcn_fused_body, B, H, BM_A),
        grid=(2, N // BM_A),
        in_specs=[
            pl.BlockSpec((BM_A, N), lambda p, j: (j, 0)),
            pl.BlockSpec((N, B * H), lambda p, j: (0, 0)),
            pl.BlockSpec((1, B * H), lambda p, j: (0, 0)),
            pl.BlockSpec((H, H), lambda p, j: (0, 0)),
            pl.BlockSpec((1, B * H), lambda p, j: (0, 0)),
            pl.BlockSpec((H, T_OUT), lambda p, j: (0, 0)),
            pl.BlockSpec((1, T_OUT), lambda p, j: (0, 0)),
        ],
        out_specs=pl.BlockSpec((B, BM_A, T_OUT), lambda p, j: (0, j, 0)),
        out_shape=jax.ShapeDtypeStruct((B, N, T_OUT), jnp.float32),
        scratch_shapes=[pltpu.VMEM((N, B * H), jnp.bfloat16)],
        compiler_params=pltpu.CompilerParams(
            dimension_semantics=("arbitrary", "arbitrary")),
    )(Abf, y1, b1t, W2.astype(jnp.bfloat16), b2t,
      Wlin.astype(jnp.bfloat16), blin.reshape(1, -1))

    return out


# speedup vs baseline: 1.0144x; 1.0144x over previous
import functools

import jax
import jax.numpy as jnp
from jax.experimental import pallas as pl
from jax.experimental.pallas import tpu as pltpu


def _leaky(x):
    return jnp.where(x >= 0, x, 0.01 * x)


def _gru_body(T, F, H, B, x_ref, wih_ref, whh_ref, bih_ref, bhh_ref, w1_ref,
              o_ref):
    x = x_ref[0]
    wih = wih_ref[...]
    whh = whh_ref[...]
    bih = bih_ref[0]
    bhh = bhh_ref[0]
    h = None
    for t in range(T):
        xt = x[:, t * F:(t + 1) * F]
        gi = jnp.dot(xt, wih, preferred_element_type=jnp.float32) + bih
        if h is None:
            gh = jnp.broadcast_to(bhh, gi.shape)
        else:
            gh = jnp.dot(h.astype(jnp.bfloat16), whh,
                         preferred_element_type=jnp.float32) + bhh
        tr = jnp.tanh(gi[:, :H] + gh[:, :H])
        tz = jnp.tanh(gi[:, H:2 * H] + gh[:, H:2 * H])
        q = gh[:, 2 * H:]
        n = jnp.tanh(gi[:, 2 * H:] + 0.5 * (tr * q + q))
        if h is None:
            h = (0.5 - 0.5 * tz) * n
        else:
            h = 0.5 * ((n + h) + tz * (h - n))
    y1 = jnp.dot(h.astype(jnp.bfloat16), w1_ref[...],
                 preferred_element_type=jnp.float32)
    o_ref[...] = y1.astype(jnp.bfloat16)


def _gcn_fused_body(B, H, BM, a_ref, y1_ref, b1_ref, w2_ref, b2_ref,
                    wlin_ref, blin_ref, o_ref, y2_ref):
    p = pl.program_id(0)
    j = pl.program_id(1)

    @pl.when(p == 0)
    def _layer1():
        u = jnp.dot(a_ref[...], y1_ref[...], preferred_element_type=jnp.float32)
        t2 = _leaky(u + b1_ref[0])
        w2 = w2_ref[...]
        for b in range(B):
            yb = jnp.dot(t2[:, b * H:(b + 1) * H].astype(jnp.bfloat16), w2,
                         preferred_element_type=jnp.float32)
            y2_ref[pl.ds(j * BM, BM), b * H:(b + 1) * H] = yb.astype(jnp.bfloat16)

    @pl.when(p == 1)
    def _layer2():
        v = jnp.dot(a_ref[...], y2_ref[...], preferred_element_type=jnp.float32)
        t3 = _leaky(v + b2_ref[0])
        wlin = wlin_ref[...]
        blin = blin_ref[0]
        for b in range(B):
            ob = jnp.dot(t3[:, b * H:(b + 1) * H].astype(jnp.bfloat16), wlin,
                         preferred_element_type=jnp.float32) + blin
            o_ref[b] = ob


def kernel(A, X, gru_Wih, gru_Whh, gru_bih, gru_bhh, W1, b1, W2, b2, Wlin, blin):
    B, N, T, F = X.shape
    H = gru_Whh.shape[1]
    T_OUT = Wlin.shape[1]

    Xr = X.reshape(B, N, T * F).astype(jnp.bfloat16)
    Abf = A.astype(jnp.bfloat16)
    sc = jnp.concatenate([jnp.full((2 * H,), 0.5, jnp.float32),
                          jnp.ones((H,), jnp.float32)])
    wih_t = (gru_Wih.T * sc).astype(jnp.bfloat16)
    whh_t = (gru_Whh.T * sc).astype(jnp.bfloat16)
    bih2 = (gru_bih * sc).reshape(1, -1)
    bhh2 = (gru_bhh * sc).reshape(1, -1)
    b1t = jnp.tile(b1, B).reshape(1, B * H)
    b2t = jnp.tile(b2, B).reshape(1, B * H)

    BM_G = min(N, 1024)
    BM_A = min(N, 256)

    y1 = pl.pallas_call(
        functools.partial(_gru_body, T, F, H, B),
        grid=(B, N // BM_G),
        in_specs=[
            pl.BlockSpec((1, BM_G, T * F), lambda b, j: (b, j, 0)),
            pl.BlockSpec((F, 3 * H), lambda b, j: (0, 0)),
            pl.BlockSpec((H, 3 * H), lambda b, j: (0, 0)),
            pl.BlockSpec((1, 3 * H), lambda b, j: (0, 0)),
            pl.BlockSpec((1, 3 * H), lambda b, j: (0, 0)),
            pl.BlockSpec((H, H), lambda b, j: (0, 0)),
        ],
        out_specs=pl.BlockSpec((BM_G, H), lambda b, j: (j, b)),
        out_shape=jax.ShapeDtypeStruct((N, B * H), jnp.bfloat16),
        compiler_params=pltpu.CompilerParams(
            dimension_semantics=("parallel", "parallel")),
    )(Xr, wih_t, whh_t, bih2, bhh2, W1.astype(jnp.bfloat16))

    out = pl.pallas_call(
        functools.partial(_gcn_fused_body, B, H, BM_A),
        grid=(2, N // BM_A),
        in_specs=[
            pl.BlockSpec((BM_A, N), lambda p, j: (j, 0)),
            pl.BlockSpec((N, B * H), lambda p, j: (0, 0)),
            pl.BlockSpec((1, B * H), lambda p, j: (0, 0)),
            pl.BlockSpec((H, H), lambda p, j: (0, 0)),
            pl.BlockSpec((1, B * H), lambda p, j: (0, 0)),
            pl.BlockSpec((H, T_OUT), lambda p, j: (0, 0)),
            pl.BlockSpec((1, T_OUT), lambda p, j: (0, 0)),
        ],
        out_specs=pl.BlockSpec((B, BM_A, T_OUT), lambda p, j: (0, j, 0)),
        out_shape=jax.ShapeDtypeStruct((B, N, T_OUT), jnp.float32),
        scratch_shapes=[pltpu.VMEM((N, B * H), jnp.bfloat16)],
        compiler_params=pltpu.CompilerParams(
            dimension_semantics=("arbitrary", "arbitrary")),
    )(Abf, y1, b1t, W2.astype(jnp.bfloat16), b2t,
      Wlin.astype(jnp.bfloat16), blin.reshape(1, -1))

    return out


# speedup vs baseline: 1.0251x; 1.0105x over previous
import functools

import jax
import jax.numpy as jnp
from jax.experimental import pallas as pl
from jax.experimental.pallas import tpu as pltpu


def _leaky(x):
    return jnp.where(x >= 0, x, 0.01 * x)


def _fused_body(T, F, H, B, BM_G, BM_A, n_jg, nb_a,
                x_ref, wih_ref, whh_ref, bih_ref, bhh_ref, w1_ref,
                a_ref, b1_ref, w2_ref, b2_ref, wlin_ref, blin_ref,
                o_ref, y1_ref, y2_ref):
    s = pl.program_id(0)
    n_gru = B * n_jg

    @pl.when(s < n_gru)
    def _gru():
        b = s // n_jg
        jg = s % n_jg
        x = x_ref[0]
        wih = wih_ref[...]
        whh = whh_ref[...]
        bih = bih_ref[0]
        bhh = bhh_ref[0]
        h = None
        for t in range(T):
            xt = x[:, t * F:(t + 1) * F]
            gi = jnp.dot(xt, wih, preferred_element_type=jnp.float32) + bih
            if h is None:
                gh = jnp.broadcast_to(bhh, gi.shape)
            else:
                gh = jnp.dot(h.astype(jnp.bfloat16), whh,
                             preferred_element_type=jnp.float32) + bhh
            r = 0.5 * jnp.tanh(0.5 * (gi[:, :H] + gh[:, :H])) + 0.5
            z = 0.5 * jnp.tanh(0.5 * (gi[:, H:2 * H] + gh[:, H:2 * H])) + 0.5
            n = jnp.tanh(gi[:, 2 * H:] + r * gh[:, 2 * H:])
            if h is None:
                h = (1.0 - z) * n
            else:
                h = (1.0 - z) * n + z * h
        y1 = jnp.dot(h.astype(jnp.bfloat16), w1_ref[...],
                     preferred_element_type=jnp.float32)
        y1_ref[pl.ds(jg * BM_G, BM_G), pl.ds(b * H, H)] = y1.astype(jnp.bfloat16)

    @pl.when((s >= n_gru) & (s < n_gru + nb_a))
    def _layer1():
        j = s - n_gru
        u = jnp.dot(a_ref[...], y1_ref[...], preferred_element_type=jnp.float32)
        t2 = _leaky(u + b1_ref[0])
        w2 = w2_ref[...]
        for b in range(B):
            yb = jnp.dot(t2[:, b * H:(b + 1) * H].astype(jnp.bfloat16), w2,
                         preferred_element_type=jnp.float32)
            y2_ref[pl.ds(j * BM_A, BM_A), b * H:(b + 1) * H] = yb.astype(jnp.bfloat16)

    @pl.when(s >= n_gru + nb_a)
    def _layer2():
        v = jnp.dot(a_ref[...], y2_ref[...], preferred_element_type=jnp.float32)
        t3 = _leaky(v + b2_ref[0])
        wlin = wlin_ref[...]
        blin = blin_ref[0]
        for b in range(B):
            ob = jnp.dot(t3[:, b * H:(b + 1) * H].astype(jnp.bfloat16), wlin,
                         preferred_element_type=jnp.float32) + blin
            o_ref[b] = ob


def kernel(A, X, gru_Wih, gru_Whh, gru_bih, gru_bhh, W1, b1, W2, b2, Wlin, blin):
    B, N, T, F = X.shape
    H = gru_Whh.shape[1]
    T_OUT = Wlin.shape[1]

    Xr = X.reshape(B, N, T * F).astype(jnp.bfloat16)
    Abf = A.astype(jnp.bfloat16)
    wih_t = gru_Wih.T.astype(jnp.bfloat16)
    whh_t = gru_Whh.T.astype(jnp.bfloat16)
    bih2 = gru_bih.reshape(1, -1)
    bhh2 = gru_bhh.reshape(1, -1)
    b1t = jnp.tile(b1, B).reshape(1, B * H)
    b2t = jnp.tile(b2, B).reshape(1, B * H)

    BM_G = min(N, 1024)
    BM_A = min(N, 256)
    n_jg = N // BM_G
    nb_a = N // BM_A
    n_gru = B * n_jg
    n_steps = n_gru + 2 * nb_a

    def _x_map(s):
        return (jnp.minimum(s // n_jg, B - 1), s % n_jg, 0)

    def _a_map(s):
        j = jnp.where(s < n_gru + nb_a, s - n_gru, s - n_gru - nb_a)
        return (jnp.clip(j, 0, nb_a - 1), 0)

    def _o_map(s):
        return (0, jnp.clip(s - n_gru - nb_a, 0, nb_a - 1), 0)

    const2 = lambda s: (0, 0)

    out = pl.pallas_call(
        functools.partial(_fused_body, T, F, H, B, BM_G, BM_A, n_jg, nb_a),
        grid=(n_steps,),
        in_specs=[
            pl.BlockSpec((1, BM_G, T * F), _x_map),
            pl.BlockSpec((F, 3 * H), const2),
            pl.BlockSpec((H, 3 * H), const2),
            pl.BlockSpec((1, 3 * H), const2),
            pl.BlockSpec((1, 3 * H), const2),
            pl.BlockSpec((H, H), const2),
            pl.BlockSpec((BM_A, N), _a_map),
            pl.BlockSpec((1, B * H), const2),
            pl.BlockSpec((H, H), const2),
            pl.BlockSpec((1, B * H), const2),
            pl.BlockSpec((H, T_OUT), const2),
            pl.BlockSpec((1, T_OUT), const2),
        ],
        out_specs=pl.BlockSpec((B, BM_A, T_OUT), _o_map),
        out_shape=jax.ShapeDtypeStruct((B, N, T_OUT), jnp.float32),
        scratch_shapes=[pltpu.VMEM((N, B * H), jnp.bfloat16),
                        pltpu.VMEM((N, B * H), jnp.bfloat16)],
        compiler_params=pltpu.CompilerParams(
            dimension_semantics=("arbitrary",)),
    )(Xr, wih_t, whh_t, bih2, bhh2, W1.astype(jnp.bfloat16),
      Abf, b1t, W2.astype(jnp.bfloat16), b2t,
      Wlin.astype(jnp.bfloat16), blin.reshape(1, -1))

    return out


# speedup vs baseline: 1.0345x; 1.0091x over previous
import functools

import jax
import jax.numpy as jnp
from jax.experimental import pallas as pl
from jax.experimental.pallas import tpu as pltpu


def _leaky(x):
    return jnp.where(x >= 0, x, 0.01 * x)


def _fused_body(T, F, H, B, BM_G, BM_A, n_jg, nb_a,
                x_ref, wih_ref, whh_ref, bih_ref, bhh_ref, w1_ref,
                a_ref, b1_ref, w2_ref, b2_ref, wlin_ref, blin_ref,
                o_ref, y1_ref, y2_ref):
    s = pl.program_id(0)
    n_gru = B * n_jg

    @pl.when(s < n_gru)
    def _gru():
        b = s // n_jg
        jg = s % n_jg
        x = x_ref[0]
        wih = wih_ref[...]
        whh = whh_ref[...]
        bih = bih_ref[0]
        bhh = bhh_ref[0]
        h = None
        for t in range(T):
            xt = x[:, t * F:(t + 1) * F]
            gi = jnp.dot(xt, wih, preferred_element_type=jnp.float32) + bih
            if h is None:
                gh = jnp.broadcast_to(bhh, gi.shape)
            else:
                gh = jnp.dot(h.astype(jnp.bfloat16), whh,
                             preferred_element_type=jnp.float32) + bhh
            r = 0.5 * jnp.tanh(0.5 * (gi[:, :H] + gh[:, :H])) + 0.5
            z = 0.5 * jnp.tanh(0.5 * (gi[:, H:2 * H] + gh[:, H:2 * H])) + 0.5
            n = jnp.tanh(gi[:, 2 * H:] + r * gh[:, 2 * H:])
            if h is None:
                h = (1.0 - z) * n
            else:
                h = (1.0 - z) * n + z * h
        y1 = jnp.dot(h.astype(jnp.bfloat16), w1_ref[...],
                     preferred_element_type=jnp.float32)
        y1_ref[pl.ds(jg * BM_G, BM_G), pl.ds(b * H, H)] = y1.astype(jnp.bfloat16)

    @pl.when((s >= n_gru) & (s < n_gru + nb_a))
    def _layer1():
        j = s - n_gru
        u = jnp.dot(a_ref[...], y1_ref[...], preferred_element_type=jnp.float32)
        t2 = _leaky(u + b1_ref[0])
        w2 = w2_ref[...]
        for b in range(B):
            yb = jnp.dot(t2[:, b * H:(b + 1) * H].astype(jnp.bfloat16), w2,
                         preferred_element_type=jnp.float32)
            y2_ref[pl.ds(j * BM_A, BM_A), b * H:(b + 1) * H] = yb.astype(jnp.bfloat16)

    @pl.when(s >= n_gru + nb_a)
    def _layer2():
        v = jnp.dot(a_ref[...], y2_ref[...], preferred_element_type=jnp.float32)
        t3 = _leaky(v + b2_ref[0])
        wlin = wlin_ref[...]
        blin = blin_ref[0]
        for b in range(B):
            ob = jnp.dot(t3[:, b * H:(b + 1) * H].astype(jnp.bfloat16), wlin,
                         preferred_element_type=jnp.float32) + blin
            o_ref[b] = ob


def kernel(A, X, gru_Wih, gru_Whh, gru_bih, gru_bhh, W1, b1, W2, b2, Wlin, blin):
    B, N, T, F = X.shape
    H = gru_Whh.shape[1]
    T_OUT = Wlin.shape[1]

    Xr = X.reshape(B, N, T * F).astype(jnp.bfloat16)
    Abf = A.astype(jnp.bfloat16)
    wih_t = gru_Wih.T.astype(jnp.bfloat16)
    whh_t = gru_Whh.T.astype(jnp.bfloat16)
    bih2 = gru_bih.reshape(1, -1)
    bhh2 = gru_bhh.reshape(1, -1)
    b1t = jnp.tile(b1, B).reshape(1, B * H)
    b2t = jnp.tile(b2, B).reshape(1, B * H)

    BM_G = min(N, 1024)
    BM_A = min(N, 512)
    n_jg = N // BM_G
    nb_a = N // BM_A
    n_gru = B * n_jg
    n_steps = n_gru + 2 * nb_a

    def _x_map(s):
        return (jnp.minimum(s // n_jg, B - 1), s % n_jg, 0)

    def _a_map(s):
        j = jnp.where(s < n_gru + nb_a, s - n_gru, s - n_gru - nb_a)
        return (jnp.clip(j, 0, nb_a - 1), 0)

    def _o_map(s):
        return (0, jnp.clip(s - n_gru - nb_a, 0, nb_a - 1), 0)

    const2 = lambda s: (0, 0)

    out = pl.pallas_call(
        functools.partial(_fused_body, T, F, H, B, BM_G, BM_A, n_jg, nb_a),
        grid=(n_steps,),
        in_specs=[
            pl.BlockSpec((1, BM_G, T * F), _x_map),
            pl.BlockSpec((F, 3 * H), const2),
            pl.BlockSpec((H, 3 * H), const2),
            pl.BlockSpec((1, 3 * H), const2),
            pl.BlockSpec((1, 3 * H), const2),
            pl.BlockSpec((H, H), const2),
            pl.BlockSpec((BM_A, N), _a_map),
            pl.BlockSpec((1, B * H), const2),
            pl.BlockSpec((H, H), const2),
            pl.BlockSpec((1, B * H), const2),
            pl.BlockSpec((H, T_OUT), const2),
            pl.BlockSpec((1, T_OUT), const2),
        ],
        out_specs=pl.BlockSpec((B, BM_A, T_OUT), _o_map),
        out_shape=jax.ShapeDtypeStruct((B, N, T_OUT), jnp.float32),
        scratch_shapes=[pltpu.VMEM((N, B * H), jnp.bfloat16),
                        pltpu.VMEM((N, B * H), jnp.bfloat16)],
        compiler_params=pltpu.CompilerParams(
            dimension_semantics=("arbitrary",)),
    )(Xr, wih_t, whh_t, bih2, bhh2, W1.astype(jnp.bfloat16),
      Abf, b1t, W2.astype(jnp.bfloat16), b2t,
      Wlin.astype(jnp.bfloat16), blin.reshape(1, -1))

    return out
